# Initial kernel scaffold; baseline (speedup 1.0000x reference)
import functools

import jax
import jax.numpy as jnp
from jax import lax
from jax.experimental import pallas as pl
from jax.experimental.pallas import tpu as pltpu
from jax.experimental.pallas import tpu_sc as plsc

N = 10000
E = 320000
D = 128
R = 100
NB = 4

NC = 2
NS = 16
NW = NC * NS

_ROWS_PER_SUB = N // NS

_MESH = plsc.VectorSubcoreMesh(core_axis_name="c", subcore_axis_name="s")


_GB = 320
_NPAD = NW * _GB


@functools.partial(
    pl.kernel,
    out_type=jax.ShapeDtypeStruct((_NPAD, D), jnp.float32),
    mesh=_MESH,
    scratch_types=[
        pltpu.VMEM((_GB,), jnp.int32),
        pltpu.VMEM((_GB, D), jnp.float32),
        pltpu.SemaphoreType.DMA,
    ],
)
def _sc_gather(table_hbm, idx_hbm, out_hbm, idx_v, rows_v, sem):
    wid = lax.axis_index("s") * NC + lax.axis_index("c")
    base = pl.multiple_of(wid * _GB, 8)
    pltpu.sync_copy(idx_hbm.at[pl.ds(base, _GB)], idx_v)
    pltpu.async_copy(table_hbm.at[idx_v], rows_v, sem).wait()
    pltpu.sync_copy(rows_v, out_hbm.at[pl.ds(base, _GB)])


_CG = 200
_CGROUPS = E // (NW * _CG)


@functools.partial(
    pl.kernel,
    out_type=jax.ShapeDtypeStruct((NC, N, 16), jnp.float32),
    mesh=_MESH,
    scratch_types=[
        pltpu.VMEM((_CG,), jnp.int32),
        pltpu.VMEM((_CG, 16), jnp.float32),
        pltpu.VMEM_SHARED((N, 16), jnp.float32),
    ],
)
def _sc_count(dst_hbm, out_hbm, dst_v, ones_v, acc):
    cid = lax.axis_index("c")
    sid = lax.axis_index("s")
    wid = sid * NC + cid

    def zfill(i, _):
        ones_v[i, :] = jnp.zeros((16,), jnp.float32)
        return 0

    lax.fori_loop(0, _CG, zfill, 0)
    base_r = sid * _ROWS_PER_SUB
    for k in range(5):
        pltpu.sync_copy(ones_v.at[pl.ds(0, 125)],
                        acc.at[pl.ds(base_r + k * 125, 125)])

    def fill(i, _):
        ones_v[i, :] = jnp.ones((16,), jnp.float32)
        return 0

    lax.fori_loop(0, _CG, fill, 0)
    plsc.subcore_barrier()

    ebase = wid * (E // NW)

    def group(g, _):
        off = pl.multiple_of(ebase + g * _CG, 8)
        pltpu.sync_copy(dst_hbm.at[pl.ds(off, _CG)], dst_v)
        pltpu.sync_copy(ones_v, acc.at[dst_v], add=True)
        return 0

    lax.fori_loop(0, _CGROUPS, group, 0)
    plsc.subcore_barrier()
    pltpu.sync_copy(acc.at[pl.ds(base_r, _ROWS_PER_SUB)],
                    out_hbm.at[cid, pl.ds(base_r, _ROWS_PER_SUB)])


_G = 80
_GROUPS = E // (NW * _G)
_ATT = NB * 128


@functools.partial(
    pl.kernel,
    out_type=jax.ShapeDtypeStruct((NC, N, D), jnp.float32),
    mesh=_MESH,
    scratch_types=[
        pltpu.VMEM((_G,), jnp.int32),
        pltpu.VMEM((_G,), jnp.int32),
        pltpu.VMEM((_G,), jnp.int32),
        pltpu.VMEM((_G,), jnp.float32),
        pltpu.VMEM((_ATT,), jnp.float32),
        pltpu.VMEM((NB, _G), jnp.float32),
        pltpu.VMEM((_G, NB * D), jnp.float32),
        pltpu.VMEM((_G, D), jnp.float32),
        pltpu.VMEM_SHARED((N, D), jnp.float32),
        pltpu.SemaphoreType.DMA,
    ],
)
def _sc_conv(src_hbm, dst_hbm, type_hbm, norm_hbm, att_hbm, y_hbm, out_hbm,
             src_v, dst_v, type_v, norm_v, att_v, c_v, ybuf, msg_v, acc, sem):
    cid = lax.axis_index("c")
    sid = lax.axis_index("s")
    wid = sid * NC + cid

    pltpu.sync_copy(att_hbm, att_v)

    def zrow(i, _):
        for dj in range(D // 16):
            msg_v[i, pl.ds(dj * 16, 16)] = jnp.zeros((16,), jnp.float32)
        return 0

    lax.fori_loop(0, _G, zrow, 0)
    base_r = sid * _ROWS_PER_SUB
    for k in range(7):
        pltpu.sync_copy(msg_v, acc.at[pl.ds(base_r + k * _G, _G)])
    pltpu.sync_copy(msg_v.at[pl.ds(0, 65)],
                    acc.at[pl.ds(base_r + 7 * _G, 65)])
    plsc.subcore_barrier()

    ebase = wid * (E // NW)

    def group(g, _):
        off = pl.multiple_of(ebase + g * _G, 8)
        pltpu.sync_copy(src_hbm.at[pl.ds(off, _G)], src_v)
        pltpu.sync_copy(dst_hbm.at[pl.ds(off, _G)], dst_v)
        pltpu.sync_copy(type_hbm.at[pl.ds(off, _G)], type_v)
        pltpu.sync_copy(norm_hbm.at[pl.ds(off, _G)], norm_v)
        cp = pltpu.async_copy(y_hbm.at[src_v], ybuf, sem)
        for j in range(_G // 16):
            t16 = type_v[pl.ds(j * 16, 16)]
            n16 = norm_v[pl.ds(j * 16, 16)]
            for b in range(NB):
                c_v[b, pl.ds(j * 16, 16)] = (
                    plsc.load_gather(att_v, [t16 + b * 128]) * n16)
        cp.wait()

        def edge(e, _):
            c0 = c_v[0, e]
            c1 = c_v[1, e]
            c2 = c_v[2, e]
            c3 = c_v[3, e]
            for dj in range(D // 16):
                v = ybuf[e, pl.ds(dj * 16, 16)] * c0
                v = v + ybuf[e, pl.ds(D + dj * 16, 16)] * c1
                v = v + ybuf[e, pl.ds(2 * D + dj * 16, 16)] * c2
                v = v + ybuf[e, pl.ds(3 * D + dj * 16, 16)] * c3
                msg_v[e, pl.ds(dj * 16, 16)] = v
            return 0

        lax.fori_loop(0, _G, edge, 0)
        pltpu.sync_copy(msg_v, acc.at[dst_v], add=True)
        return 0

    lax.fori_loop(0, _GROUPS, group, 0)
    plsc.subcore_barrier()
    pltpu.sync_copy(acc.at[pl.ds(base_r, _ROWS_PER_SUB)],
                    out_hbm.at[cid, pl.ds(base_r, _ROWS_PER_SUB)])


_BM = 400


def _mm_body(x_ref, w_ref, root_ref, bias_ref, y_ref, xr_ref):
    xb = x_ref[...]
    y_ref[...] = jnp.dot(xb, w_ref[...], preferred_element_type=jnp.float32)
    xr_ref[...] = (jnp.dot(xb, root_ref[...], preferred_element_type=jnp.float32)
                   + bias_ref[...])


def _tc_matmul(x, wcat, root, bias):
    return pl.pallas_call(
        _mm_body,
        grid=(N // _BM,),
        in_specs=[
            pl.BlockSpec((_BM, D), lambda i: (i, 0)),
            pl.BlockSpec((D, NB * D), lambda i: (0, 0)),
            pl.BlockSpec((D, D), lambda i: (0, 0)),
            pl.BlockSpec((1, D), lambda i: (0, 0)),
        ],
        out_specs=[
            pl.BlockSpec((_BM, NB * D), lambda i: (i, 0)),
            pl.BlockSpec((_BM, D), lambda i: (i, 0)),
        ],
        out_shape=[
            jax.ShapeDtypeStruct((N, NB * D), jnp.float32),
            jax.ShapeDtypeStruct((N, D), jnp.float32),
        ],
    )(x, wcat, root, bias)


def _combine_body(relu, p_ref, cnt_ref, xr_ref, o_ref):
    cnt = cnt_ref[0, :, 0:1] + cnt_ref[1, :, 0:1]
    inv = 1.0 / jnp.maximum(cnt, 1.0)
    out = (p_ref[0] + p_ref[1]) * inv + xr_ref[...]
    if relu:
        out = jnp.maximum(out, 0.0)
    o_ref[...] = out


def _tc_combine(p, cntp, xr, relu):
    return pl.pallas_call(
        functools.partial(_combine_body, relu),
        grid=(N // _BM,),
        in_specs=[
            pl.BlockSpec((NC, _BM, D), lambda i: (0, i, 0)),
            pl.BlockSpec((NC, _BM, 16), lambda i: (0, i, 0)),
            pl.BlockSpec((_BM, D), lambda i: (i, 0)),
        ],
        out_specs=pl.BlockSpec((_BM, D), lambda i: (i, 0)),
        out_shape=jax.ShapeDtypeStruct((N, D), jnp.float32),
    )(p, cntp, xr)


def _att_flat(att):
    return jnp.pad(att.T, ((0, 0), (0, 128 - R))).reshape(-1)


def _layer(x, basis, att, root, bias, src, dst, etype, norm, cntp, relu):
    wcat = basis.transpose(1, 0, 2).reshape(D, NB * D)
    y, xr = _tc_matmul(x, wcat, root, bias)
    p = _sc_conv(src, dst, etype, norm, _att_flat(att), y)
    return _tc_combine(p, cntp, xr, relu)


def kernel(entity, edge_index, edge_type, edge_norm, entity_table,
           basis1, att1, root1, bias1, basis2, att2, root2, bias2):
    src = edge_index[0].astype(jnp.int32)
    dst = edge_index[1].astype(jnp.int32)
    etype = edge_type.astype(jnp.int32)
    norm = edge_norm.astype(jnp.float32)

    ent = jnp.concatenate(
        [entity.astype(jnp.int32),
         jnp.zeros((_NPAD - N,), jnp.int32)])
    x = _sc_gather(entity_table, ent)[:N]
    cntp = _sc_count(dst)

    b1 = bias1.reshape(1, D)
    b2 = bias2.reshape(1, D)
    x = _layer(x, basis1, att1, root1, b1, src, dst, etype, norm, cntp, False)
    x = _layer(x, basis1, att1, root1, b1, src, dst, etype, norm, cntp, True)
    x = _layer(x, basis2, att2, root2, b2, src, dst, etype, norm, cntp, False)
    return x


# speedup vs baseline: 6.6808x; 6.6808x over previous
import functools

import jax
import jax.numpy as jnp
from jax import lax
from jax.experimental import pallas as pl
from jax.experimental.pallas import tpu as pltpu
from jax.experimental.pallas import tpu_sc as plsc

N = 10000
E = 320000
D = 128
R = 100
NB = 4

NC = 2
NS = 16
NW = NC * NS

NACC = 10240
_ROWS_PER_SUB = NACC // NS

_MESH = plsc.VectorSubcoreMesh(core_axis_name="c", subcore_axis_name="s")


_GB = 320
_NPAD = NW * _GB


@functools.partial(
    pl.kernel,
    out_type=jax.ShapeDtypeStruct((_NPAD, D), jnp.float32),
    mesh=_MESH,
    scratch_types=[
        pltpu.VMEM((_GB,), jnp.int32),
        pltpu.VMEM((_GB, D), jnp.float32),
        pltpu.SemaphoreType.DMA,
    ],
)
def _sc_gather(table_hbm, idx_hbm, out_hbm, idx_v, rows_v, sem):
    wid = lax.axis_index("s") * NC + lax.axis_index("c")
    base = pl.multiple_of(wid * _GB, 8)
    pltpu.sync_copy(idx_hbm.at[pl.ds(base, _GB)], idx_v)
    pltpu.async_copy(table_hbm.at[idx_v], rows_v, sem).wait()
    pltpu.sync_copy(rows_v, out_hbm.at[pl.ds(base, _GB)])


_CG = 128
_CNGT = E // _CG
_CNGQ = _CNGT // NW
_CNGR = _CNGT % NW


@functools.partial(
    pl.kernel,
    out_type=jax.ShapeDtypeStruct((NC, NACC, 16), jnp.float32),
    mesh=_MESH,
    scratch_types=[
        pltpu.VMEM((_CG,), jnp.int32),
        pltpu.VMEM((_CG, 16), jnp.float32),
        pltpu.VMEM_SHARED((NACC, 16), jnp.float32),
    ],
)
def _sc_count(dst_hbm, out_hbm, dst_v, ones_v, acc):
    cid = lax.axis_index("c")
    sid = lax.axis_index("s")
    wid = sid * NC + cid

    def zfill(i, _):
        ones_v[i, :] = jnp.zeros((16,), jnp.float32)
        return 0

    lax.fori_loop(0, _CG, zfill, 0)
    base_r = sid * _ROWS_PER_SUB
    for k in range(5):
        pltpu.sync_copy(ones_v.at[pl.ds(0, 128)],
                        acc.at[pl.ds(base_r + k * 128, 128)])

    def fill(i, _):
        ones_v[i, :] = jnp.ones((16,), jnp.float32)
        return 0

    lax.fori_loop(0, _CG, fill, 0)
    plsc.subcore_barrier()

    gbase = wid * _CNGQ + jnp.minimum(wid, _CNGR)
    ngroups = _CNGQ + (wid < _CNGR).astype(jnp.int32)

    def group(g, _):
        off = pl.multiple_of((gbase + g) * _CG, 8)
        pltpu.sync_copy(dst_hbm.at[pl.ds(off, _CG)], dst_v)
        pltpu.sync_copy(ones_v, acc.at[dst_v], add=True)
        return 0

    lax.fori_loop(0, ngroups, group, 0)
    plsc.subcore_barrier()
    pltpu.sync_copy(acc.at[pl.ds(base_r, _ROWS_PER_SUB)],
                    out_hbm.at[cid, pl.ds(base_r, _ROWS_PER_SUB)])


_G = 64
_NGT = E // _G
_NGQ = _NGT // NW
_NGR = _NGT % NW
_ATT = NB * 128


@functools.partial(
    pl.kernel,
    out_type=jax.ShapeDtypeStruct((NC, NACC, D), jnp.float32),
    mesh=_MESH,
    scratch_types=[
        pltpu.VMEM((_G,), jnp.int32),
        pltpu.VMEM((_G,), jnp.int32),
        pltpu.VMEM((_G,), jnp.int32),
        pltpu.VMEM((_G,), jnp.float32),
        pltpu.VMEM((_ATT,), jnp.float32),
        pltpu.SMEM((_ATT,), jnp.float32),
        pltpu.VMEM((_G, NB * D), jnp.float32),
        pltpu.VMEM((_G, D), jnp.float32),
        pltpu.VMEM_SHARED((NACC, D), jnp.float32),
        pltpu.SemaphoreType.DMA,
    ],
)
def _sc_conv(src_hbm, dst_hbm, type_hbm, norm_hbm, att_hbm, y_hbm, out_hbm,
             src_v, dst_v, type_v, norm_v, att_v, att_s, ybuf, msg_v, acc, sem):
    cid = lax.axis_index("c")
    sid = lax.axis_index("s")
    wid = sid * NC + cid

    pltpu.sync_copy(att_hbm, att_v)

    def afill(i, _):
        chunk = att_v[pl.ds(i * 16, 16)]
        for k in range(16):
            att_s[i * 16 + k] = chunk[k]
        return 0

    lax.fori_loop(0, _ATT // 16, afill, 0)

    def zrow(i, _):
        for dj in range(D // 16):
            msg_v[i, pl.ds(dj * 16, 16)] = jnp.zeros((16,), jnp.float32)
        return 0

    lax.fori_loop(0, _G, zrow, 0)
    base_r = sid * _ROWS_PER_SUB
    for k in range(10):
        pltpu.sync_copy(msg_v, acc.at[pl.ds(base_r + k * _G, _G)])
    plsc.subcore_barrier()

    gbase = wid * _NGQ + jnp.minimum(wid, _NGR)
    ngroups = _NGQ + (wid < _NGR).astype(jnp.int32)

    def group(g, _):
        off = pl.multiple_of((gbase + g) * _G, 8)
        pltpu.sync_copy(src_hbm.at[pl.ds(off, _G)], src_v)
        pltpu.sync_copy(dst_hbm.at[pl.ds(off, _G)], dst_v)
        pltpu.sync_copy(type_hbm.at[pl.ds(off, _G)], type_v)
        pltpu.sync_copy(norm_hbm.at[pl.ds(off, _G)], norm_v)
        cp = pltpu.async_copy(y_hbm.at[src_v], ybuf, sem)
        cp.wait()

        def subgroup(j, _):
            tch = type_v[pl.ds(j * 16, 16)]
            nch = norm_v[pl.ds(j * 16, 16)]
            for k in range(16):
                e = j * 16 + k
                t = tch[k]
                n = nch[k]
                c0 = att_s[t] * n
                c1 = att_s[t + 128] * n
                c2 = att_s[t + 256] * n
                c3 = att_s[t + 384] * n
                for dj in range(D // 16):
                    v = ybuf[e, pl.ds(dj * 16, 16)] * c0
                    v = v + ybuf[e, pl.ds(D + dj * 16, 16)] * c1
                    v = v + ybuf[e, pl.ds(2 * D + dj * 16, 16)] * c2
                    v = v + ybuf[e, pl.ds(3 * D + dj * 16, 16)] * c3
                    msg_v[e, pl.ds(dj * 16, 16)] = v
            return 0

        lax.fori_loop(0, _G // 16, subgroup, 0)
        pltpu.sync_copy(msg_v, acc.at[dst_v], add=True)
        return 0

    lax.fori_loop(0, ngroups, group, 0)
    plsc.subcore_barrier()
    pltpu.sync_copy(acc.at[pl.ds(base_r, _ROWS_PER_SUB)],
                    out_hbm.at[cid, pl.ds(base_r, _ROWS_PER_SUB)])


_BM = 400


def _mm_body(x_ref, w_ref, root_ref, bias_ref, y_ref, xr_ref):
    xb = x_ref[...]
    y_ref[...] = jnp.dot(xb, w_ref[...], preferred_element_type=jnp.float32)
    xr_ref[...] = (jnp.dot(xb, root_ref[...], preferred_element_type=jnp.float32)
                   + bias_ref[...])


def _tc_matmul(x, wcat, root, bias):
    return pl.pallas_call(
        _mm_body,
        grid=(N // _BM,),
        in_specs=[
            pl.BlockSpec((_BM, D), lambda i: (i, 0)),
            pl.BlockSpec((D, NB * D), lambda i: (0, 0)),
            pl.BlockSpec((D, D), lambda i: (0, 0)),
            pl.BlockSpec((1, D), lambda i: (0, 0)),
        ],
        out_specs=[
            pl.BlockSpec((_BM, NB * D), lambda i: (i, 0)),
            pl.BlockSpec((_BM, D), lambda i: (i, 0)),
        ],
        out_shape=[
            jax.ShapeDtypeStruct((N, NB * D), jnp.float32),
            jax.ShapeDtypeStruct((N, D), jnp.float32),
        ],
    )(x, wcat, root, bias)


def _combine_body(relu, p_ref, cnt_ref, xr_ref, o_ref):
    cnt = cnt_ref[0, :, 0:1] + cnt_ref[1, :, 0:1]
    inv = 1.0 / jnp.maximum(cnt, 1.0)
    out = (p_ref[0] + p_ref[1]) * inv + xr_ref[...]
    if relu:
        out = jnp.maximum(out, 0.0)
    o_ref[...] = out


def _tc_combine(p, cntp, xr, relu):
    return pl.pallas_call(
        functools.partial(_combine_body, relu),
        grid=(N // _BM,),
        in_specs=[
            pl.BlockSpec((NC, _BM, D), lambda i: (0, i, 0)),
            pl.BlockSpec((NC, _BM, 16), lambda i: (0, i, 0)),
            pl.BlockSpec((_BM, D), lambda i: (i, 0)),
        ],
        out_specs=pl.BlockSpec((_BM, D), lambda i: (i, 0)),
        out_shape=jax.ShapeDtypeStruct((N, D), jnp.float32),
    )(p, cntp, xr)


def _att_flat(att):
    return jnp.pad(att.T, ((0, 0), (0, 128 - R))).reshape(-1)


def _layer(x, basis, att, root, bias, src, dst, etype, norm, cntp, relu):
    wcat = basis.transpose(1, 0, 2).reshape(D, NB * D)
    y, xr = _tc_matmul(x, wcat, root, bias)
    if _DEBUG_XLA_CONV:
        coef = jnp.take(att, etype, axis=0) * norm[:, None]
        msg = (jnp.take(y, src, axis=0).reshape(E, NB, D)
               * coef[:, :, None]).sum(axis=1)
        summed = jax.ops.segment_sum(msg, dst, num_segments=NACC)
        p = jnp.stack([summed, jnp.zeros_like(summed)])
    else:
        p = _sc_conv(src, dst, etype, norm, _att_flat(att), y)
    return _tc_combine(p, cntp, xr, relu)


_DEBUG_XLA_CONV = False
_DEBUG_XLA_G = False
_DEBUG_XLA_C = False


def kernel(entity, edge_index, edge_type, edge_norm, entity_table,
           basis1, att1, root1, bias1, basis2, att2, root2, bias2):
    src = edge_index[0].astype(jnp.int32)
    dst = edge_index[1].astype(jnp.int32)
    etype = edge_type.astype(jnp.int32)
    norm = edge_norm.astype(jnp.float32)

    ent = jnp.concatenate(
        [entity.astype(jnp.int32),
         jnp.zeros((_NPAD - N,), jnp.int32)])
    if _DEBUG_XLA_G:
        x = jnp.take(entity_table, entity, axis=0)
    else:
        x = _sc_gather(entity_table, ent)[:N]
    if _DEBUG_XLA_C:
        cnt = jax.ops.segment_sum(jnp.ones((E,), jnp.float32), dst,
                                  num_segments=NACC)
        cntp = jnp.stack([jnp.tile(cnt[:, None], (1, 16)),
                          jnp.zeros((NACC, 16), jnp.float32)])
    else:
        cntp = _sc_count(dst)

    b1 = bias1.reshape(1, D)
    b2 = bias2.reshape(1, D)
    x = _layer(x, basis1, att1, root1, b1, src, dst, etype, norm, cntp, False)
    x = _layer(x, basis1, att1, root1, b1, src, dst, etype, norm, cntp, True)
    x = _layer(x, basis2, att2, root2, b2, src, dst, etype, norm, cntp, False)
    return x
